# baseline (device time: 130891 ns/iter reference)
import jax
import jax.numpy as jnp
from jax import lax
from jax.experimental import pallas as pl
from jax.experimental.pallas import tpu as pltpu

N_DEV = 4
N_TOK = 2048
D = 1024
H = 1024
N_EXP = 32
EXP_PER_DEV = N_EXP // N_DEV
CAP = 51
CAP_PAD = 64
ROWS = EXP_PER_DEV * CAP_PAD
CHUNK = N_TOK // N_DEV


def _moe_matmul(xg, expert_W):

    def body(x_ref, w_ref, y_ref):
        y_ref[0] = jnp.dot(x_ref[0], w_ref[0], preferred_element_type=jnp.float32)

    return pl.pallas_call(
        body,
        grid=(EXP_PER_DEV,),
        in_specs=[
            pl.BlockSpec((1, CAP_PAD, D), lambda e: (e, 0, 0)),
            pl.BlockSpec((1, D, H), lambda e: (e, 0, 0)),
        ],
        out_specs=pl.BlockSpec((1, CAP_PAD, H), lambda e: (e, 0, 0)),
        out_shape=jax.ShapeDtypeStruct((EXP_PER_DEV, CAP_PAD, H), jnp.float32),
    )(xg, expert_W)


def _ring_reduce_scatter(partial):

    def body(p_ref, out_ref, comm_ref, send_sems, recv_sems):
        d = lax.axis_index("i")
        left = lax.rem(d + N_DEV - 1, N_DEV)
        right = lax.rem(d + 1, N_DEV)

        barrier = pltpu.get_barrier_semaphore()
        for nbr in (left, right):
            pl.semaphore_signal(
                barrier, inc=1, device_id=(nbr,),
                device_id_type=pl.DeviceIdType.MESH,
            )
        pl.semaphore_wait(barrier, 2)

        c0 = lax.rem(d + N_DEV - 1, N_DEV)
        comm_ref[0] = p_ref[pl.ds(c0 * CHUNK, CHUNK), :]
        for s in range(N_DEV - 1):
            rdma = pltpu.make_async_remote_copy(
                src_ref=comm_ref.at[s],
                dst_ref=comm_ref.at[s + 1],
                send_sem=send_sems.at[s],
                recv_sem=recv_sems.at[s],
                device_id=(right,),
                device_id_type=pl.DeviceIdType.MESH,
            )
            rdma.start()
            rdma.wait()
            c = lax.rem(d + 2 * N_DEV - s - 2, N_DEV)
            comm_ref[s + 1] = comm_ref[s + 1] + p_ref[pl.ds(c * CHUNK, CHUNK), :]
        out_ref[...] = comm_ref[N_DEV - 1]

    return pl.pallas_call(
        body,
        out_shape=jax.ShapeDtypeStruct((CHUNK, H), jnp.float32),
        in_specs=[pl.BlockSpec(memory_space=pltpu.VMEM)],
        out_specs=pl.BlockSpec(memory_space=pltpu.VMEM),
        scratch_shapes=[
            pltpu.VMEM((N_DEV, CHUNK, H), jnp.float32),
            pltpu.SemaphoreType.DMA((N_DEV - 1,)),
            pltpu.SemaphoreType.DMA((N_DEV - 1,)),
        ],
        compiler_params=pltpu.CompilerParams(collective_id=0),
    )(partial)


def kernel(x, router_W, route_idx, expert_W):
    del router_W
    pos = lax.axis_index("i")

    e = route_idx[:, 0].astype(jnp.int32)
    oh = (e[:, None] == jnp.arange(N_EXP, dtype=jnp.int32)[None, :]).astype(jnp.int32)
    rank = jnp.take_along_axis(jnp.cumsum(oh, axis=0), e[:, None], axis=1)[:, 0] - 1

    local_e = e - EXP_PER_DEV * pos
    mine = (local_e >= 0) & (local_e < EXP_PER_DEV) & (rank < CAP)
    tokens = jnp.arange(N_TOK, dtype=jnp.int32)
    flat = jnp.where(mine, local_e * CAP_PAD + rank, ROWS)
    slot_tok = jnp.zeros((ROWS + 1,), jnp.int32).at[flat].set(tokens)[:ROWS]
    slot_valid = jnp.zeros((ROWS + 1,), jnp.bool_).at[flat].set(True)[:ROWS]

    xg = x[slot_tok].reshape(EXP_PER_DEV, CAP_PAD, D)
    y = _moe_matmul(xg, expert_W).reshape(ROWS, H)

    dst = jnp.where(slot_valid, slot_tok, N_TOK)
    partial = jnp.zeros((N_TOK, H), jnp.float32).at[dst].set(y, mode="drop")

    return _ring_reduce_scatter(partial)
